# Initial kernel scaffold; baseline (speedup 1.0000x reference)
import jax
import jax.numpy as jnp
from jax.experimental import pallas as pl


def kernel(in_feat, edge_index, edge_weight, W1, b1, W2, b2):
    raise NotImplementedError("write your pallas kernel here")


# speedup vs baseline: 13.4508x; 13.4508x over previous
import functools

import jax
import jax.numpy as jnp
from jax import lax
from jax.experimental import pallas as pl
from jax.experimental.pallas import tpu as pltpu
from jax.experimental.pallas import tpu_sc as plsc

N = 10000
E = 320000
D_IN = 128
D_H = 128
N_CLS = 64

NC = 2
NS = 16
NW = NC * NS
L = 16

NPAD = 10240
EPT = E // NW
CW = 80
NCHK = EPT // CW
NSTG = 5
SCHK = NCHK // NSTG
NBLK = N // CW
BPT = 8
DP = 128

_mesh = lambda: plsc.VectorSubcoreMesh(core_axis_name="c", subcore_axis_name="s")


@functools.partial(
    pl.kernel,
    out_type=[jax.ShapeDtypeStruct((NW, 1, NPAD // NS), jnp.float32),
              jax.ShapeDtypeStruct((NW, 1, NPAD // NS), jnp.float32)],
    mesh=_mesh(),
    scratch_types=[
        pltpu.VMEM((NCHK, CW), jnp.int32),
        pltpu.VMEM((NCHK, CW), jnp.int32),
        pltpu.VMEM((CW,), jnp.float32),
        pltpu.VMEM_SHARED((NPAD,), jnp.float32),
        pltpu.VMEM_SHARED((NPAD,), jnp.float32),
        pltpu.VMEM((NPAD // NS,), jnp.float32),
    ],
)
def _deg(src_hbm, dst_hbm, dsrc_out, ddst_out,
         src_v, dst_v, ones_v, dsrc_sh, ddst_sh, zb_v):
    c = lax.axis_index("c")
    s = lax.axis_index("s")
    wid = s * NC + c
    for i in range(CW // L):
        ones_v[pl.ds(i * L, L)] = jnp.ones((L,), jnp.float32)
    for i in range((NPAD // NS) // L):
        zb_v[pl.ds(i * L, L)] = jnp.zeros((L,), jnp.float32)
    sl = NPAD // NS
    pltpu.sync_copy(zb_v, dsrc_sh.at[pl.ds(s * sl, sl)])
    pltpu.sync_copy(zb_v, ddst_sh.at[pl.ds(s * sl, sl)])
    pltpu.sync_copy(src_hbm.at[wid], src_v)
    pltpu.sync_copy(dst_hbm.at[wid], dst_v)
    plsc.subcore_barrier()

    def body(j, carry):
        pltpu.sync_copy(ones_v, dsrc_sh.at[src_v.at[j]], add=True)
        pltpu.sync_copy(ones_v, ddst_sh.at[dst_v.at[j]], add=True)
        return carry

    lax.fori_loop(0, NCHK, body, 0)
    plsc.subcore_barrier()
    row = c * NS + s
    pltpu.sync_copy(dsrc_sh.at[pl.ds(s * sl, sl)], zb_v)
    pltpu.sync_copy(zb_v, dsrc_out.at[row, 0])
    pltpu.sync_copy(ddst_sh.at[pl.ds(s * sl, sl)], zb_v)
    pltpu.sync_copy(zb_v, ddst_out.at[row, 0])


@functools.partial(
    pl.kernel,
    out_type=jax.ShapeDtypeStruct((NC, NBLK, CW, DP), jnp.float32),
    mesh=_mesh(),
    scratch_types=[
        pltpu.VMEM((SCHK, CW), jnp.int32),
        pltpu.VMEM((SCHK, CW), jnp.int32),
        pltpu.VMEM((SCHK, CW), jnp.float32),
        pltpu.VMEM((CW, DP), jnp.float32),
        pltpu.VMEM((CW, DP), jnp.float32),
        pltpu.VMEM_SHARED((N, DP), jnp.float32),
        pltpu.SemaphoreType.DMA,
        pltpu.SemaphoreType.DMA,
    ],
)
def _mp(h_hbm, src_hbm, dst_hbm, ew_hbm, out_hbm,
        src_v, dst_v, ew_v, rows0_v, rows1_v, acc_sh, sem0, sem1):
    c = lax.axis_index("c")
    s = lax.axis_index("s")
    wid = s * NC + c

    def zrow(r, carry):
        for k in range(DP // L):
            rows0_v[r, pl.ds(k * L, L)] = jnp.zeros((L,), jnp.float32)
        return carry

    lax.fori_loop(0, CW, zrow, 0)
    for t in range(BPT):
        @pl.when(s * BPT + t < NBLK)
        def _():
            pltpu.sync_copy(rows0_v, acc_sh.at[pl.ds((s * BPT + t) * CW, CW)])
    plsc.subcore_barrier()

    def _scale(rows_v, j):
        def gbody(g, inner):
            ew16 = ew_v[j, pl.ds(g * L, L)]
            for e in range(L):
                cwv = ew16.at[jnp.full((L,), e, jnp.int32)].get(
                    mode="promise_in_bounds")
                r = g * L + e
                for k in range(DP // L):
                    rows_v[r, pl.ds(k * L, L)] = (
                        rows_v[r, pl.ds(k * L, L)] * cwv)
            return inner

        lax.fori_loop(0, CW // L, gbody, 0)

    for q in range(NSTG):
        pltpu.sync_copy(src_hbm.at[wid, q], src_v)
        pltpu.sync_copy(dst_hbm.at[wid, q], dst_v)
        pltpu.sync_copy(ew_hbm.at[wid, q], ew_v)
        pltpu.async_copy(h_hbm.at[src_v.at[0]], rows0_v, sem0)

        def pair(jj, carry):
            j0 = jj * 2
            pltpu.make_async_copy(
                h_hbm.at[src_v.at[j0]], rows0_v, sem0).wait()
            pltpu.async_copy(h_hbm.at[src_v.at[j0 + 1]], rows1_v, sem1)
            _scale(rows0_v, j0)
            pltpu.sync_copy(rows0_v, acc_sh.at[dst_v.at[j0]], add=True)
            pltpu.make_async_copy(
                h_hbm.at[src_v.at[j0 + 1]], rows1_v, sem1).wait()

            @pl.when(j0 + 2 < SCHK)
            def _():
                pltpu.async_copy(h_hbm.at[src_v.at[j0 + 2]], rows0_v, sem0)

            _scale(rows1_v, j0 + 1)
            pltpu.sync_copy(rows1_v, acc_sh.at[dst_v.at[j0 + 1]], add=True)
            return carry

        lax.fori_loop(0, SCHK // 2, pair, 0)
        pltpu.make_async_copy(
            h_hbm.at[src_v.at[SCHK - 1]], rows0_v, sem0).wait()
        _scale(rows0_v, SCHK - 1)
        pltpu.sync_copy(rows0_v, acc_sh.at[dst_v.at[SCHK - 1]], add=True)

    plsc.subcore_barrier()
    for t in range(BPT):
        @pl.when(s * BPT + t < NBLK)
        def _():
            blk = s * BPT + t
            pltpu.sync_copy(acc_sh.at[pl.ds(blk * CW, CW)], rows0_v)
            pltpu.sync_copy(rows0_v, out_hbm.at[c, blk])


def _norms_body(ds_ref, dd_ref, ns_ref, nd_ref):
    ds = ds_ref[0] + ds_ref[1]
    dd = dd_ref[0] + dd_ref[1]
    ns_ref[...] = jnp.where(ds > 0, lax.rsqrt(ds), 0.0)
    nd_ref[...] = jnp.where(dd > 0, lax.rsqrt(dd), 0.0)


def _mm1_body(x_ref, w_ref, ns_ref, o_ref):
    h = jnp.dot(x_ref[...], w_ref[...], preferred_element_type=jnp.float32)
    o_ref[...] = h * ns_ref[...]


def _mid_body(a0_ref, a1_ref, nd_ref, b1_ref, w2_ref, ns_ref, o_ref):
    agg = (a0_ref[...] + a1_ref[...]) * nd_ref[...] + b1_ref[...]
    z = jnp.maximum(agg, 0.0)
    h2 = jnp.dot(z, w2_ref[...], preferred_element_type=jnp.float32)
    o_ref[...] = h2 * ns_ref[...]


def _fin_body(a0_ref, a1_ref, nd_ref, b2_ref, o_ref):
    x = (a0_ref[...] + a1_ref[...]) * nd_ref[...] + b2_ref[...]
    m = jnp.max(x, axis=1, keepdims=True)
    ex = jnp.exp(x - m)
    o_ref[...] = ex / jnp.sum(ex, axis=1, keepdims=True)


_RB = 2000
_G = N // _RB


def _tc_norms(dsrc, ddst):
    return pl.pallas_call(
        _norms_body,
        out_shape=[jax.ShapeDtypeStruct((NPAD // 128, 128), jnp.float32)] * 2,
    )(dsrc, ddst)


def _tc_mm1(x, W1, ns_col):
    return pl.pallas_call(
        _mm1_body,
        grid=(_G,),
        in_specs=[pl.BlockSpec((_RB, D_IN), lambda i: (i, 0)),
                  pl.BlockSpec((D_IN, D_H), lambda i: (0, 0)),
                  pl.BlockSpec((_RB, 1), lambda i: (i, 0))],
        out_specs=pl.BlockSpec((_RB, D_H), lambda i: (i, 0)),
        out_shape=jax.ShapeDtypeStruct((N, D_H), jnp.float32),
    )(x, W1, ns_col)


def _tc_mid(a0, a1, nd_col, b1r, W2p, ns_col):
    return pl.pallas_call(
        _mid_body,
        grid=(_G,),
        in_specs=[pl.BlockSpec((_RB, D_H), lambda i: (i, 0)),
                  pl.BlockSpec((_RB, D_H), lambda i: (i, 0)),
                  pl.BlockSpec((_RB, 1), lambda i: (i, 0)),
                  pl.BlockSpec((1, D_H), lambda i: (0, 0)),
                  pl.BlockSpec((D_H, DP), lambda i: (0, 0)),
                  pl.BlockSpec((_RB, 1), lambda i: (i, 0))],
        out_specs=pl.BlockSpec((_RB, DP), lambda i: (i, 0)),
        out_shape=jax.ShapeDtypeStruct((N, DP), jnp.float32),
    )(a0, a1, nd_col, b1r, W2p, ns_col)


def _tc_fin(a0, a1, nd_col, b2r):
    return pl.pallas_call(
        _fin_body,
        grid=(_G,),
        in_specs=[pl.BlockSpec((_RB, N_CLS), lambda i: (i, 0)),
                  pl.BlockSpec((_RB, N_CLS), lambda i: (i, 0)),
                  pl.BlockSpec((_RB, 1), lambda i: (i, 0)),
                  pl.BlockSpec((1, N_CLS), lambda i: (0, 0))],
        out_specs=pl.BlockSpec((_RB, N_CLS), lambda i: (i, 0)),
        out_shape=jax.ShapeDtypeStruct((N, N_CLS), jnp.float32),
    )(a0, a1, nd_col, b2r)


def kernel(in_feat, edge_index, edge_weight, W1, b1, W2, b2):
    src3 = edge_index[0].reshape(NW, NCHK, CW)
    dst3 = edge_index[1].reshape(NW, NCHK, CW)
    src4 = edge_index[0].reshape(NW, NSTG, SCHK, CW)
    dst4 = edge_index[1].reshape(NW, NSTG, SCHK, CW)
    ew4 = edge_weight.reshape(NW, NSTG, SCHK, CW)

    dsrc, ddst = _deg(src3, dst3)
    ns2, nd2 = _tc_norms(dsrc.reshape(NC, NPAD // 128, 128),
                         ddst.reshape(NC, NPAD // 128, 128))
    ns_col = ns2.reshape(NPAD, 1)[:N]
    nd_col = nd2.reshape(NPAD, 1)[:N]

    h1s = _tc_mm1(in_feat, W1, ns_col)
    agg1 = _mp(h1s, src4, dst4, ew4).reshape(NC, N, D_H)
    W2p = jnp.concatenate(
        [W2, jnp.zeros((D_H, DP - N_CLS), jnp.float32)], axis=1)
    h2s = _tc_mid(agg1[0], agg1[1], nd_col, b1.reshape(1, D_H), W2p, ns_col)
    agg2 = _mp(h2s, src4, dst4, ew4).reshape(NC, N, DP)
    return _tc_fin(agg2[0, :, :N_CLS], agg2[1, :, :N_CLS], nd_col,
                   b2.reshape(1, N_CLS))
